# Initial kernel scaffold; baseline (speedup 1.0000x reference)
import jax
import jax.numpy as jnp
from jax.experimental import pallas as pl


def kernel(p, f, pe, W1_0, g1_0, b1_0, W1_1, g1_1, b1_1, W2_0, g2_0, b2_0, W2_1, g2_1, b2_1, W2_2, g2_2, b2_2):
    raise NotImplementedError("write your pallas kernel here")


# speedup vs baseline: 10.0458x; 10.0458x over previous
import functools

import numpy as np
import jax
import jax.numpy as jnp
from jax import lax
from jax.experimental import pallas as pl
from jax.experimental.pallas import tpu as pltpu
from jax.experimental.pallas import tpu_sc as plsc

B, N, IN_C, OUT_C = 2, 8192, 32, 64
STRIDE, NSAMPLE = 4, 32
M = N // STRIDE
BMS = B * M * NSAMPLE
R2 = np.float32(0.1 ** 2)
EPS = np.float32(1e-5)


def _stats_update(y, s_ref, q_ref, step):
    @pl.when(step == 0)
    def _():
        s_ref[...] = jnp.zeros_like(s_ref)
        q_ref[...] = jnp.zeros_like(q_ref)

    s_ref[...] += jnp.sum(y, axis=0, keepdims=True)
    q_ref[...] += jnp.sum(y * y, axis=0, keepdims=True)


def _affine(s_in, q_in, g_ref, b_ref, ntot):
    mean = s_in[...] * np.float32(1.0 / ntot)
    var = q_in[...] * np.float32(1.0 / ntot) - mean * mean
    a = g_ref[...] / jnp.sqrt(var + EPS)
    c = b_ref[...] - mean * a
    return a, c


def _mm_stats_body(x_ref, w_ref, out_ref, s_ref, q_ref):
    y = lax.dot_general(x_ref[...], w_ref[...], (((1,), (1,)), ((), ())),
                        preferred_element_type=jnp.float32)
    out_ref[...] = y
    _stats_update(y, s_ref, q_ref, pl.program_id(0))


def _bn_mm_stats_body(x_ref, s_in, q_in, g_ref, b_ref, w_ref,
                      out_ref, s_ref, q_ref, *, ntot):
    a, c = _affine(s_in, q_in, g_ref, b_ref, ntot)
    h = jnp.maximum(x_ref[...] * a + c, 0.0)
    y = lax.dot_general(h, w_ref[...], (((1,), (1,)), ((), ())),
                        preferred_element_type=jnp.float32)
    out_ref[...] = y
    _stats_update(y, s_ref, q_ref, pl.program_id(0))


def _bn_relu_body(x_ref, s_in, q_in, g_ref, b_ref, out_ref, *, ntot):
    a, c = _affine(s_in, q_in, g_ref, b_ref, ntot)
    out_ref[...] = jnp.maximum(x_ref[...] * a + c, 0.0)


def _dp_mm_stats_body(gx_ref, np_ref, w_ref, out_ref, s_ref, q_ref):
    dp = gx_ref[:, 0:3] - np_ref[:, 0:3]
    y = lax.dot_general(dp, w_ref[...], (((1,), (1,)), ((), ())),
                        preferred_element_type=jnp.float32)
    out_ref[...] = y
    _stats_update(y, s_ref, q_ref, pl.program_id(0))


def _pe_max_body(x_ref, s_in, q_in, g_ref, b_ref, fj_ref,
                 pe_ref, fout_ref, *, ntot):
    a, c = _affine(s_in, q_in, g_ref, b_ref, ntot)
    pe = jnp.maximum(x_ref[...] * a + c, 0.0)
    pe_ref[...] = pe
    tot = pe + fj_ref[...]
    rt = tot.shape[0]
    fout_ref[...] = jnp.max(tot.reshape(rt // NSAMPLE, NSAMPLE, tot.shape[1]),
                            axis=1)


def _row_spec(rt, cols):
    return pl.BlockSpec((rt, cols), lambda i: (i, 0))


def _fix_spec(rows, cols):
    return pl.BlockSpec((rows, cols), lambda i: (0, 0))


def _stats_shapes(cout):
    return (jax.ShapeDtypeStruct((1, cout), jnp.float32),
            jax.ShapeDtypeStruct((1, cout), jnp.float32))


def _mm_stats(x, w, rt):
    rows, cout = x.shape[0], w.shape[0]
    return pl.pallas_call(
        _mm_stats_body,
        grid=(rows // rt,),
        in_specs=[_row_spec(rt, x.shape[1]), _fix_spec(*w.shape)],
        out_specs=(_row_spec(rt, cout), _fix_spec(1, cout), _fix_spec(1, cout)),
        out_shape=(jax.ShapeDtypeStruct((rows, cout), jnp.float32),
                   *_stats_shapes(cout)),
    )(x, w)


def _bn_mm_stats(x, s, q, g, b, w, rt):
    rows, cin, cout = x.shape[0], x.shape[1], w.shape[0]
    return pl.pallas_call(
        functools.partial(_bn_mm_stats_body, ntot=rows),
        grid=(rows // rt,),
        in_specs=[_row_spec(rt, cin), _fix_spec(1, cin), _fix_spec(1, cin),
                  _fix_spec(1, cin), _fix_spec(1, cin), _fix_spec(*w.shape)],
        out_specs=(_row_spec(rt, cout), _fix_spec(1, cout), _fix_spec(1, cout)),
        out_shape=(jax.ShapeDtypeStruct((rows, cout), jnp.float32),
                   *_stats_shapes(cout)),
    )(x, s, q, g, b, w)


def _bn_relu(x, s, q, g, b, rt):
    rows, cin = x.shape
    return pl.pallas_call(
        functools.partial(_bn_relu_body, ntot=rows),
        grid=(rows // rt,),
        in_specs=[_row_spec(rt, cin), _fix_spec(1, cin), _fix_spec(1, cin),
                  _fix_spec(1, cin), _fix_spec(1, cin)],
        out_specs=_row_spec(rt, cin),
        out_shape=jax.ShapeDtypeStruct((rows, cin), jnp.float32),
    )(x, s, q, g, b)


def _dp_mm_stats(gxyz, npb, w, rt):
    rows, cout = gxyz.shape[0], w.shape[0]
    return pl.pallas_call(
        _dp_mm_stats_body,
        grid=(rows // rt,),
        in_specs=[_row_spec(rt, 16), _row_spec(rt, 16), _fix_spec(*w.shape)],
        out_specs=(_row_spec(rt, cout), _fix_spec(1, cout), _fix_spec(1, cout)),
        out_shape=(jax.ShapeDtypeStruct((rows, cout), jnp.float32),
                   *_stats_shapes(cout)),
    )(gxyz, npb, w)


def _pe_max(x, s, q, g, b, fj, rt):
    rows, cin = x.shape
    return pl.pallas_call(
        functools.partial(_pe_max_body, ntot=rows),
        grid=(rows // rt,),
        in_specs=[_row_spec(rt, cin), _fix_spec(1, cin), _fix_spec(1, cin),
                  _fix_spec(1, cin), _fix_spec(1, cin), _row_spec(rt, cin)],
        out_specs=(_row_spec(rt, cin),
                   pl.BlockSpec((rt // NSAMPLE, cin), lambda i: (i, 0))),
        out_shape=(jax.ShapeDtypeStruct((rows, cin), jnp.float32),
                   jax.ShapeDtypeStruct((rows // NSAMPLE, cin), jnp.float32)),
    )(x, s, q, g, b, fj)


_MT = 256
_NC = 256
_NCH = N // _NC


def _ballq_body(pT_ref, np_ref, gidx_ref, counts_scr, cnt_scr):
    nc = pl.program_id(2)

    @pl.when(nc == 0)
    def _():
        counts_scr[...] = jnp.zeros_like(counts_scr)
        cnt_scr[...] = jnp.zeros_like(cnt_scr)

    qx = np_ref[:, 0:1]
    qy = np_ref[:, 1:2]
    qz = np_ref[:, 2:3]
    px = pT_ref[0:1, :]
    py = pT_ref[1:2, :]
    pz = pT_ref[2:3, :]
    dx = qx - px
    dy = qy - py
    dz = qz - pz
    d2 = (dx * dx + dy * dy) + dz * dz
    wf = jnp.where(d2 < R2, 1.0, 0.0).astype(jnp.float32)

    ii = lax.broadcasted_iota(jnp.int32, (_NC, _NC), 0)
    jj = lax.broadcasted_iota(jnp.int32, (_NC, _NC), 1)
    ut = jnp.where(ii <= jj, 1.0, 0.0).astype(jnp.float32)
    rank = cnt_scr[...] + lax.dot_general(
        wf, ut, (((1,), (0,)), ((), ())), preferred_element_type=jnp.float32)

    cols = [jnp.sum(jnp.where(rank <= np.float32(s), 1.0, 0.0),
                    axis=1, keepdims=True) for s in range(NSAMPLE)]
    counts_scr[...] += jnp.concatenate(cols, axis=1)
    cnt_scr[...] += jnp.sum(wf, axis=1, keepdims=True)

    @pl.when(nc == _NCH - 1)
    def _():
        cnt = cnt_scr[...]
        counts = counts_scr[...]
        first = jnp.where(cnt > 0, counts[:, 0:1], 0.0)
        siota = lax.broadcasted_iota(
            jnp.int32, (_MT, NSAMPLE), 1).astype(jnp.float32)
        g = jnp.where(siota < cnt, counts, first)
        gidx_ref[...] = g.astype(jnp.int32)


def _ball_query(pT, np16):
    return pl.pallas_call(
        _ballq_body,
        grid=(B, M // _MT, _NCH),
        in_specs=[
            pl.BlockSpec((None, 3, _NC), lambda b, mt, nc: (b, 0, nc)),
            pl.BlockSpec((None, _MT, 16), lambda b, mt, nc: (b, mt, 0)),
        ],
        out_specs=pl.BlockSpec((None, _MT, NSAMPLE), lambda b, mt, nc: (b, mt, 0)),
        out_shape=jax.ShapeDtypeStruct((B, M, NSAMPLE), jnp.int32),
        scratch_shapes=[pltpu.VMEM((_MT, NSAMPLE), jnp.float32),
                        pltpu.VMEM((_MT, 1), jnp.float32)],
    )(pT, np16)


_NW = 32


def _sc_gather_centroids(p16, iflat):
    rows = iflat.shape[0]
    per_w = rows // _NW
    mesh = plsc.VectorSubcoreMesh(core_axis_name="c", subcore_axis_name="s")

    @functools.partial(
        pl.kernel, mesh=mesh,
        out_type=jax.ShapeDtypeStruct((rows, 16), jnp.float32),
        scratch_types=[pltpu.VMEM((per_w,), jnp.int32),
                       pltpu.VMEM((per_w, 16), jnp.float32),
                       pltpu.SemaphoreType.DMA],
        compiler_params=pltpu.CompilerParams(use_tc_tiling_on_sc=False),
    )
    def k(tab_hbm, idx_hbm, out_hbm, idx_v, rows_v, sem):
        wid = lax.axis_index("s") * 2 + lax.axis_index("c")
        base = wid * per_w
        pltpu.sync_copy(idx_hbm.at[pl.ds(base, per_w)], idx_v)
        pltpu.async_copy(tab_hbm.at[idx_v], rows_v, sem).wait()
        pltpu.sync_copy(rows_v, out_hbm.at[pl.ds(base, per_w)])

    return k(p16, iflat)


def _sc_gather_groups(f1t, p16, iflat):
    rows = iflat.shape[0]
    per_w = rows // _NW
    chunk = 1024
    nch = per_w // chunk
    mesh = plsc.VectorSubcoreMesh(core_axis_name="c", subcore_axis_name="s")

    @functools.partial(
        pl.kernel, mesh=mesh,
        out_type=(jax.ShapeDtypeStruct((rows, OUT_C), jnp.float32),
                  jax.ShapeDtypeStruct((rows, 16), jnp.float32)),
        scratch_types=[pltpu.VMEM((chunk,), jnp.int32),
                       pltpu.VMEM((chunk, OUT_C), jnp.float32),
                       pltpu.VMEM((chunk, 16), jnp.float32),
                       pltpu.SemaphoreType.DMA,
                       pltpu.SemaphoreType.DMA],
        compiler_params=pltpu.CompilerParams(use_tc_tiling_on_sc=False),
    )
    def k(f_hbm, p_hbm, idx_hbm, fj_hbm, gx_hbm, idx_v, f_v, x_v, sem1, sem2):
        wid = lax.axis_index("s") * 2 + lax.axis_index("c")
        base = wid * per_w
        for j in range(nch):
            off = base + j * chunk
            pltpu.sync_copy(idx_hbm.at[pl.ds(off, chunk)], idx_v)
            cp1 = pltpu.async_copy(f_hbm.at[idx_v], f_v, sem1)
            cp2 = pltpu.async_copy(p_hbm.at[idx_v], x_v, sem2)
            cp1.wait()
            cp2.wait()
            pltpu.sync_copy(f_v, fj_hbm.at[pl.ds(off, chunk)])
            pltpu.sync_copy(x_v, gx_hbm.at[pl.ds(off, chunk)])

    return k(f1t, p16, iflat)


def kernel(p, f, pe, W1_0, g1_0, b1_0, W1_1, g1_1, b1_1, W2_0, g2_0, b2_0,
           W2_1, g2_1, b2_1, W2_2, g2_2, b2_2):
    del pe
    r2 = lambda v: v.reshape(1, -1)

    idx = jax.random.randint(jax.random.key(42), (B, M), 0, N)
    offs = (jnp.arange(B, dtype=jnp.int32) * N)[:, None]
    iflat_c = (idx.astype(jnp.int32) + offs).reshape(-1)

    p16 = jnp.zeros((B * N, 16), jnp.float32).at[:, 0:3].set(p.reshape(B * N, 3))
    fT2 = jnp.transpose(f, (0, 2, 1)).reshape(B * N, IN_C)
    pT = jnp.transpose(p, (0, 2, 1))

    h0, s0, q0 = _mm_stats(fT2, W1_0, 2048)
    h1, s1, q1 = _bn_mm_stats(h0, s0, q0, r2(g1_0), r2(b1_0), W1_1, 2048)
    f1t = _bn_relu(h1, s1, q1, r2(g1_1), r2(b1_1), 2048)

    np16_flat = _sc_gather_centroids(p16, iflat_c)
    np16 = np16_flat.reshape(B, M, 16)
    new_p = np16[:, :, 0:3]

    gidx = _ball_query(pT, np16)

    iflat_g = (gidx + offs[:, :, None]).reshape(-1)
    fj_flat, gx_flat = _sc_gather_groups(f1t, p16, iflat_g)

    npb = jnp.broadcast_to(np16_flat[:, None, :], (B * M, NSAMPLE, 16))
    npb = npb.reshape(BMS, 16)
    h1p, t1, u1 = _dp_mm_stats(gx_flat, npb, W2_0, 4096)
    h2p, t2, u2 = _bn_mm_stats(h1p, t1, u1, r2(g2_0), r2(b2_0), W2_1, 4096)
    h3p, t3, u3 = _bn_mm_stats(h2p, t2, u2, r2(g2_1), r2(b2_1), W2_2, 4096)
    pe_flat, fout_flat = _pe_max(h3p, t3, u3, r2(g2_2), r2(b2_2), fj_flat, 4096)

    pe_out = pe_flat.reshape(B, M, NSAMPLE, OUT_C).transpose(0, 3, 1, 2)
    fout = fout_flat.reshape(B, M, OUT_C).transpose(0, 2, 1)
    return new_p, fout, pe_out


# speedup vs baseline: 10.2237x; 1.0177x over previous
import functools

import numpy as np
import jax
import jax.numpy as jnp
from jax import lax
from jax.experimental import pallas as pl
from jax.experimental.pallas import tpu as pltpu
from jax.experimental.pallas import tpu_sc as plsc

B, N, IN_C, OUT_C = 2, 8192, 32, 64
STRIDE, NSAMPLE = 4, 32
M = N // STRIDE
BMS = B * M * NSAMPLE
R2 = np.float32(0.1 ** 2)
EPS = np.float32(1e-5)


def _stats_update(y, s_ref, q_ref, step):
    @pl.when(step == 0)
    def _():
        s_ref[...] = jnp.zeros_like(s_ref)
        q_ref[...] = jnp.zeros_like(q_ref)

    ones = jnp.ones((1, y.shape[0]), jnp.float32)
    s_ref[...] += lax.dot_general(ones, y, (((1,), (0,)), ((), ())),
                                  preferred_element_type=jnp.float32)
    q_ref[...] += lax.dot_general(ones, y * y, (((1,), (0,)), ((), ())),
                                  preferred_element_type=jnp.float32)


def _affine(s_in, q_in, g_ref, b_ref, ntot):
    mean = s_in[...] * np.float32(1.0 / ntot)
    var = q_in[...] * np.float32(1.0 / ntot) - mean * mean
    a = g_ref[...] / jnp.sqrt(var + EPS)
    c = b_ref[...] - mean * a
    return a, c


def _mm_stats_body(x_ref, w_ref, out_ref, s_ref, q_ref):
    y = lax.dot_general(x_ref[...], w_ref[...], (((1,), (1,)), ((), ())),
                        preferred_element_type=jnp.float32)
    out_ref[...] = y
    _stats_update(y, s_ref, q_ref, pl.program_id(0))


def _bn_mm_stats_body(x_ref, s_in, q_in, g_ref, b_ref, w_ref,
                      out_ref, s_ref, q_ref, *, ntot):
    a, c = _affine(s_in, q_in, g_ref, b_ref, ntot)
    h = jnp.maximum(x_ref[...] * a + c, 0.0)
    y = lax.dot_general(h, w_ref[...], (((1,), (1,)), ((), ())),
                        preferred_element_type=jnp.float32)
    out_ref[...] = y
    _stats_update(y, s_ref, q_ref, pl.program_id(0))


def _bn_relu_body(x_ref, s_in, q_in, g_ref, b_ref, out_ref, *, ntot):
    a, c = _affine(s_in, q_in, g_ref, b_ref, ntot)
    out_ref[...] = jnp.maximum(x_ref[...] * a + c, 0.0)


def _dp_mm_stats_body(gx_ref, np_ref, w_ref, out_ref, s_ref, q_ref):
    dp = gx_ref[:, 0:3] - np_ref[:, 0:3]
    y = lax.dot_general(dp, w_ref[...], (((1,), (1,)), ((), ())),
                        preferred_element_type=jnp.float32)
    out_ref[...] = y
    _stats_update(y, s_ref, q_ref, pl.program_id(0))


def _pe_max_body(x_ref, s_in, q_in, g_ref, b_ref, fj_ref,
                 pe_ref, fout_ref, *, ntot):
    a, c = _affine(s_in, q_in, g_ref, b_ref, ntot)
    pe = jnp.maximum(x_ref[...] * a + c, 0.0)
    pe_ref[...] = pe
    tot = pe + fj_ref[...]
    rt = tot.shape[0]
    fout_ref[...] = jnp.max(tot.reshape(rt // NSAMPLE, NSAMPLE, tot.shape[1]),
                            axis=1)


def _row_spec(rt, cols):
    return pl.BlockSpec((rt, cols), lambda i: (i, 0))


def _fix_spec(rows, cols):
    return pl.BlockSpec((rows, cols), lambda i: (0, 0))


def _stats_shapes(cout):
    return (jax.ShapeDtypeStruct((1, cout), jnp.float32),
            jax.ShapeDtypeStruct((1, cout), jnp.float32))


def _mm_stats(x, w, rt):
    rows, cout = x.shape[0], w.shape[0]
    return pl.pallas_call(
        _mm_stats_body,
        grid=(rows // rt,),
        in_specs=[_row_spec(rt, x.shape[1]), _fix_spec(*w.shape)],
        out_specs=(_row_spec(rt, cout), _fix_spec(1, cout), _fix_spec(1, cout)),
        out_shape=(jax.ShapeDtypeStruct((rows, cout), jnp.float32),
                   *_stats_shapes(cout)),
    )(x, w)


def _bn_mm_stats(x, s, q, g, b, w, rt):
    rows, cin, cout = x.shape[0], x.shape[1], w.shape[0]
    return pl.pallas_call(
        functools.partial(_bn_mm_stats_body, ntot=rows),
        grid=(rows // rt,),
        in_specs=[_row_spec(rt, cin), _fix_spec(1, cin), _fix_spec(1, cin),
                  _fix_spec(1, cin), _fix_spec(1, cin), _fix_spec(*w.shape)],
        out_specs=(_row_spec(rt, cout), _fix_spec(1, cout), _fix_spec(1, cout)),
        out_shape=(jax.ShapeDtypeStruct((rows, cout), jnp.float32),
                   *_stats_shapes(cout)),
    )(x, s, q, g, b, w)


def _bn_relu(x, s, q, g, b, rt):
    rows, cin = x.shape
    return pl.pallas_call(
        functools.partial(_bn_relu_body, ntot=rows),
        grid=(rows // rt,),
        in_specs=[_row_spec(rt, cin), _fix_spec(1, cin), _fix_spec(1, cin),
                  _fix_spec(1, cin), _fix_spec(1, cin)],
        out_specs=_row_spec(rt, cin),
        out_shape=jax.ShapeDtypeStruct((rows, cin), jnp.float32),
    )(x, s, q, g, b)


def _dp_mm_stats(gxyz, npb, w, rt):
    rows, cout = gxyz.shape[0], w.shape[0]
    return pl.pallas_call(
        _dp_mm_stats_body,
        grid=(rows // rt,),
        in_specs=[_row_spec(rt, 16), _row_spec(rt, 16), _fix_spec(*w.shape)],
        out_specs=(_row_spec(rt, cout), _fix_spec(1, cout), _fix_spec(1, cout)),
        out_shape=(jax.ShapeDtypeStruct((rows, cout), jnp.float32),
                   *_stats_shapes(cout)),
    )(gxyz, npb, w)


def _pe_max(x, s, q, g, b, fj, rt):
    rows, cin = x.shape
    return pl.pallas_call(
        functools.partial(_pe_max_body, ntot=rows),
        grid=(rows // rt,),
        in_specs=[_row_spec(rt, cin), _fix_spec(1, cin), _fix_spec(1, cin),
                  _fix_spec(1, cin), _fix_spec(1, cin), _row_spec(rt, cin)],
        out_specs=(_row_spec(rt, cin),
                   pl.BlockSpec((rt // NSAMPLE, cin), lambda i: (i, 0))),
        out_shape=(jax.ShapeDtypeStruct((rows, cin), jnp.float32),
                   jax.ShapeDtypeStruct((rows // NSAMPLE, cin), jnp.float32)),
    )(x, s, q, g, b, fj)


_MT = 256
_NC = 512
_NCH = N // _NC


def _ballq_body(pT_ref, np_ref, ut_ref, gidx_ref, counts_scr, cnt_scr):
    nc = pl.program_id(2)

    @pl.when(nc == 0)
    def _():
        counts_scr[...] = jnp.zeros_like(counts_scr)
        cnt_scr[...] = jnp.zeros_like(cnt_scr)

    qx = np_ref[:, 0:1]
    qy = np_ref[:, 1:2]
    qz = np_ref[:, 2:3]
    px = pT_ref[0:1, :]
    py = pT_ref[1:2, :]
    pz = pT_ref[2:3, :]
    dx = qx - px
    dy = qy - py
    dz = qz - pz
    d2 = (dx * dx + dy * dy) + dz * dz
    wf = jnp.where(d2 < R2, 1.0, 0.0).astype(jnp.bfloat16)

    rank = cnt_scr[...] + lax.dot_general(
        wf, ut_ref[...], (((1,), (0,)), ((), ())),
        preferred_element_type=jnp.float32)
    rc = jnp.minimum(rank, 33.0).astype(jnp.bfloat16)
    onesv = jnp.ones((_NC, 1), jnp.bfloat16)
    cols = [lax.dot_general(
        jnp.where(rc <= jnp.bfloat16(s), jnp.bfloat16(1), jnp.bfloat16(0)),
        onesv, (((1,), (0,)), ((), ())), preferred_element_type=jnp.float32)
        for s in range(NSAMPLE)]
    counts_scr[...] += jnp.concatenate(cols, axis=1)
    cnt_scr[...] += lax.dot_general(wf, onesv, (((1,), (0,)), ((), ())),
                                    preferred_element_type=jnp.float32)

    @pl.when(nc == _NCH - 1)
    def _():
        cnt = cnt_scr[...]
        counts = counts_scr[...]
        first = jnp.where(cnt > 0, counts[:, 0:1], 0.0)
        siota = lax.broadcasted_iota(
            jnp.int32, (_MT, NSAMPLE), 1).astype(jnp.float32)
        g = jnp.where(siota < cnt, counts, first)
        gidx_ref[...] = g.astype(jnp.int32)


def _ball_query(pT, np16, ut):
    return pl.pallas_call(
        _ballq_body,
        grid=(B, M // _MT, _NCH),
        in_specs=[
            pl.BlockSpec((None, 3, _NC), lambda b, mt, nc: (b, 0, nc)),
            pl.BlockSpec((None, _MT, 16), lambda b, mt, nc: (b, mt, 0)),
            pl.BlockSpec((_NC, _NC), lambda b, mt, nc: (0, 0)),
        ],
        out_specs=pl.BlockSpec((None, _MT, NSAMPLE), lambda b, mt, nc: (b, mt, 0)),
        out_shape=jax.ShapeDtypeStruct((B, M, NSAMPLE), jnp.int32),
        scratch_shapes=[pltpu.VMEM((_MT, NSAMPLE), jnp.float32),
                        pltpu.VMEM((_MT, 1), jnp.float32)],
    )(pT, np16, ut)


_NW = 32


def _sc_gather_centroids(p16, iflat):
    rows = iflat.shape[0]
    per_w = rows // _NW
    mesh = plsc.VectorSubcoreMesh(core_axis_name="c", subcore_axis_name="s")

    @functools.partial(
        pl.kernel, mesh=mesh,
        out_type=jax.ShapeDtypeStruct((rows, 16), jnp.float32),
        scratch_types=[pltpu.VMEM((per_w,), jnp.int32),
                       pltpu.VMEM((per_w, 16), jnp.float32),
                       pltpu.SemaphoreType.DMA],
        compiler_params=pltpu.CompilerParams(use_tc_tiling_on_sc=False),
    )
    def k(tab_hbm, idx_hbm, out_hbm, idx_v, rows_v, sem):
        wid = lax.axis_index("s") * 2 + lax.axis_index("c")
        base = wid * per_w
        pltpu.sync_copy(idx_hbm.at[pl.ds(base, per_w)], idx_v)
        pltpu.async_copy(tab_hbm.at[idx_v], rows_v, sem).wait()
        pltpu.sync_copy(rows_v, out_hbm.at[pl.ds(base, per_w)])

    return k(p16, iflat)


def _sc_gather_groups(f1t, p16, iflat):
    rows = iflat.shape[0]
    per_w = rows // _NW
    chunk = 1024
    nch = per_w // chunk
    mesh = plsc.VectorSubcoreMesh(core_axis_name="c", subcore_axis_name="s")

    @functools.partial(
        pl.kernel, mesh=mesh,
        out_type=(jax.ShapeDtypeStruct((rows, OUT_C), jnp.float32),
                  jax.ShapeDtypeStruct((rows, 16), jnp.float32)),
        scratch_types=[pltpu.VMEM((chunk,), jnp.int32),
                       pltpu.VMEM((chunk, OUT_C), jnp.float32),
                       pltpu.VMEM((chunk, 16), jnp.float32),
                       pltpu.SemaphoreType.DMA,
                       pltpu.SemaphoreType.DMA],
        compiler_params=pltpu.CompilerParams(use_tc_tiling_on_sc=False),
    )
    def k(f_hbm, p_hbm, idx_hbm, fj_hbm, gx_hbm, idx_v, f_v, x_v, sem1, sem2):
        wid = lax.axis_index("s") * 2 + lax.axis_index("c")
        base = wid * per_w
        for j in range(nch):
            off = base + j * chunk
            pltpu.sync_copy(idx_hbm.at[pl.ds(off, chunk)], idx_v)
            cp1 = pltpu.async_copy(f_hbm.at[idx_v], f_v, sem1)
            cp2 = pltpu.async_copy(p_hbm.at[idx_v], x_v, sem2)
            cp1.wait()
            cp2.wait()
            pltpu.sync_copy(f_v, fj_hbm.at[pl.ds(off, chunk)])
            pltpu.sync_copy(x_v, gx_hbm.at[pl.ds(off, chunk)])

    return k(f1t, p16, iflat)


def kernel(p, f, pe, W1_0, g1_0, b1_0, W1_1, g1_1, b1_1, W2_0, g2_0, b2_0,
           W2_1, g2_1, b2_1, W2_2, g2_2, b2_2):
    del pe
    r2 = lambda v: v.reshape(1, -1)

    idx = jax.random.randint(jax.random.key(42), (B, M), 0, N)
    offs = (jnp.arange(B, dtype=jnp.int32) * N)[:, None]
    iflat_c = (idx.astype(jnp.int32) + offs).reshape(-1)

    p16 = jnp.zeros((B * N, 16), jnp.float32).at[:, 0:3].set(p.reshape(B * N, 3))
    fT2 = jnp.transpose(f, (0, 2, 1)).reshape(B * N, IN_C)
    pT = jnp.transpose(p, (0, 2, 1))

    h0, s0, q0 = _mm_stats(fT2, W1_0, 2048)
    h1, s1, q1 = _bn_mm_stats(h0, s0, q0, r2(g1_0), r2(b1_0), W1_1, 2048)
    f1t = _bn_relu(h1, s1, q1, r2(g1_1), r2(b1_1), 2048)

    np16_flat = _sc_gather_centroids(p16, iflat_c)
    np16 = np16_flat.reshape(B, M, 16)
    new_p = np16[:, :, 0:3]

    ii = np.arange(_NC)
    ut = jnp.asarray((ii[:, None] <= ii[None, :]), jnp.bfloat16)
    gidx = _ball_query(pT, np16, ut)

    iflat_g = (gidx + offs[:, :, None]).reshape(-1)
    fj_flat, gx_flat = _sc_gather_groups(f1t, p16, iflat_g)

    npb = jnp.broadcast_to(np16_flat[:, None, :], (B * M, NSAMPLE, 16))
    npb = npb.reshape(BMS, 16)
    h1p, t1, u1 = _dp_mm_stats(gx_flat, npb, W2_0, 4096)
    h2p, t2, u2 = _bn_mm_stats(h1p, t1, u1, r2(g2_0), r2(b2_0), W2_1, 4096)
    h3p, t3, u3 = _bn_mm_stats(h2p, t2, u2, r2(g2_1), r2(b2_1), W2_2, 4096)
    pe_flat, fout_flat = _pe_max(h3p, t3, u3, r2(g2_2), r2(b2_2), fj_flat, 4096)

    pe_out = pe_flat.reshape(B, M, NSAMPLE, OUT_C).transpose(0, 3, 1, 2)
    fout = fout_flat.reshape(B, M, OUT_C).transpose(0, 2, 1)
    return new_p, fout, pe_out


# speedup vs baseline: 17.0108x; 1.6639x over previous
import functools

import numpy as np
import jax
import jax.numpy as jnp
from jax import lax
from jax.experimental import pallas as pl
from jax.experimental.pallas import tpu as pltpu
from jax.experimental.pallas import tpu_sc as plsc

B, N, IN_C, OUT_C = 2, 8192, 32, 64
STRIDE, NSAMPLE = 4, 32
M = N // STRIDE
BMS = B * M * NSAMPLE
R2 = np.float32(0.1 ** 2)
EPS = np.float32(1e-5)


def _stats_update(y, s_ref, q_ref, step):
    @pl.when(step == 0)
    def _():
        s_ref[...] = jnp.zeros_like(s_ref)
        q_ref[...] = jnp.zeros_like(q_ref)

    ones = jnp.ones((1, y.shape[0]), jnp.float32)
    s_ref[...] += lax.dot_general(ones, y, (((1,), (0,)), ((), ())),
                                  preferred_element_type=jnp.float32)
    q_ref[...] += lax.dot_general(ones, y * y, (((1,), (0,)), ((), ())),
                                  preferred_element_type=jnp.float32)


def _affine(s_in, q_in, g_ref, b_ref, ntot):
    mean = s_in[...] * np.float32(1.0 / ntot)
    var = q_in[...] * np.float32(1.0 / ntot) - mean * mean
    a = g_ref[...] / jnp.sqrt(var + EPS)
    c = b_ref[...] - mean * a
    return a, c


def _mm_stats_body(x_ref, w_ref, out_ref, s_ref, q_ref):
    y = lax.dot_general(x_ref[...], w_ref[...], (((1,), (1,)), ((), ())),
                        preferred_element_type=jnp.float32)
    out_ref[...] = y
    _stats_update(y, s_ref, q_ref, pl.program_id(0))


def _bn_mm_stats_body(x_ref, s_in, q_in, g_ref, b_ref, w_ref,
                      out_ref, s_ref, q_ref, *, ntot):
    a, c = _affine(s_in, q_in, g_ref, b_ref, ntot)
    h = jnp.maximum(x_ref[...] * a + c, 0.0)
    y = lax.dot_general(h, w_ref[...], (((1,), (1,)), ((), ())),
                        preferred_element_type=jnp.float32)
    out_ref[...] = y
    _stats_update(y, s_ref, q_ref, pl.program_id(0))


def _bn_relu_body(x_ref, s_in, q_in, g_ref, b_ref, out_ref, *, ntot):
    a, c = _affine(s_in, q_in, g_ref, b_ref, ntot)
    out_ref[...] = jnp.maximum(x_ref[...] * a + c, 0.0)


def _dp_mm_stats_body(gx_ref, np_ref, w_ref, out_ref, s_ref, q_ref):
    dp = gx_ref[:, 0:3] - np_ref[:, 0:3]
    y = lax.dot_general(dp, w_ref[...], (((1,), (1,)), ((), ())),
                        preferred_element_type=jnp.float32)
    out_ref[...] = y
    _stats_update(y, s_ref, q_ref, pl.program_id(0))


def _pe_max_body(x_ref, s_in, q_in, g_ref, b_ref, fj_ref,
                 pe_ref, fout_ref, *, ntot):
    a, c = _affine(s_in, q_in, g_ref, b_ref, ntot)
    pe = jnp.maximum(x_ref[...] * a + c, 0.0)
    pe_ref[...] = pe
    tot = pe + fj_ref[...]
    rt = tot.shape[0]
    fout_ref[...] = jnp.max(tot.reshape(rt // NSAMPLE, NSAMPLE, tot.shape[1]),
                            axis=1)


def _row_spec(rt, cols):
    return pl.BlockSpec((rt, cols), lambda i: (i, 0))


def _fix_spec(rows, cols):
    return pl.BlockSpec((rows, cols), lambda i: (0, 0))


def _stats_shapes(cout):
    return (jax.ShapeDtypeStruct((1, cout), jnp.float32),
            jax.ShapeDtypeStruct((1, cout), jnp.float32))


def _mm_stats(x, w, rt):
    rows, cout = x.shape[0], w.shape[0]
    return pl.pallas_call(
        _mm_stats_body,
        grid=(rows // rt,),
        in_specs=[_row_spec(rt, x.shape[1]), _fix_spec(*w.shape)],
        out_specs=(_row_spec(rt, cout), _fix_spec(1, cout), _fix_spec(1, cout)),
        out_shape=(jax.ShapeDtypeStruct((rows, cout), jnp.float32),
                   *_stats_shapes(cout)),
    )(x, w)


def _bn_mm_stats(x, s, q, g, b, w, rt):
    rows, cin, cout = x.shape[0], x.shape[1], w.shape[0]
    return pl.pallas_call(
        functools.partial(_bn_mm_stats_body, ntot=rows),
        grid=(rows // rt,),
        in_specs=[_row_spec(rt, cin), _fix_spec(1, cin), _fix_spec(1, cin),
                  _fix_spec(1, cin), _fix_spec(1, cin), _fix_spec(*w.shape)],
        out_specs=(_row_spec(rt, cout), _fix_spec(1, cout), _fix_spec(1, cout)),
        out_shape=(jax.ShapeDtypeStruct((rows, cout), jnp.float32),
                   *_stats_shapes(cout)),
    )(x, s, q, g, b, w)


def _bn_relu(x, s, q, g, b, rt):
    rows, cin = x.shape
    return pl.pallas_call(
        functools.partial(_bn_relu_body, ntot=rows),
        grid=(rows // rt,),
        in_specs=[_row_spec(rt, cin), _fix_spec(1, cin), _fix_spec(1, cin),
                  _fix_spec(1, cin), _fix_spec(1, cin)],
        out_specs=_row_spec(rt, cin),
        out_shape=jax.ShapeDtypeStruct((rows, cin), jnp.float32),
    )(x, s, q, g, b)


def _dp_mm_stats(gxyz, npb, w, rt):
    rows, cout = gxyz.shape[0], w.shape[0]
    return pl.pallas_call(
        _dp_mm_stats_body,
        grid=(rows // rt,),
        in_specs=[_row_spec(rt, 16), _row_spec(rt, 16), _fix_spec(*w.shape)],
        out_specs=(_row_spec(rt, cout), _fix_spec(1, cout), _fix_spec(1, cout)),
        out_shape=(jax.ShapeDtypeStruct((rows, cout), jnp.float32),
                   *_stats_shapes(cout)),
    )(gxyz, npb, w)


def _pe_max(x, s, q, g, b, fj, rt):
    rows, cin = x.shape
    return pl.pallas_call(
        functools.partial(_pe_max_body, ntot=rows),
        grid=(rows // rt,),
        in_specs=[_row_spec(rt, cin), _fix_spec(1, cin), _fix_spec(1, cin),
                  _fix_spec(1, cin), _fix_spec(1, cin), _row_spec(rt, cin)],
        out_specs=(_row_spec(rt, cin),
                   pl.BlockSpec((rt // NSAMPLE, cin), lambda i: (i, 0))),
        out_shape=(jax.ShapeDtypeStruct((rows, cin), jnp.float32),
                   jax.ShapeDtypeStruct((rows // NSAMPLE, cin), jnp.float32)),
    )(x, s, q, g, b, fj)


_MT = 256
_NC = 512
_NCH = N // _NC
_WPC = _NC // 16
_W = N // 16
_WP = _W + 16


def _ballq_body(pT_ref, np_ref, seg_ref, p2m_ref, ut_ref,
                beta_ref, a2_ref, pw2_ref, hscr, pwscr):
    nc = pl.program_id(2)
    qx = np_ref[:, 0:1]
    qy = np_ref[:, 1:2]
    qz = np_ref[:, 2:3]
    px = pT_ref[0:1, :]
    py = pT_ref[1:2, :]
    pz = pT_ref[2:3, :]
    dx = qx - px
    dy = qy - py
    dz = qz - pz
    d2 = (dx * dx + dy * dy) + dz * dz
    wf = jnp.where(d2 < R2, 1.0, 0.0).astype(jnp.bfloat16)

    hscr[nc] = lax.dot_general(wf, seg_ref[...], (((1,), (0,)), ((), ())),
                               preferred_element_type=jnp.float32)
    pwscr[nc] = lax.dot_general(wf, p2m_ref[...], (((1,), (0,)), ((), ())),
                                preferred_element_type=jnp.float32)

    @pl.when(nc == _NCH - 1)
    def _():
        hfull = jnp.concatenate([hscr[k] for k in range(_NCH)], axis=1)
        pwfull = jnp.concatenate([pwscr[k] for k in range(_NCH)], axis=1)
        cumi = lax.dot_general(hfull.astype(jnp.bfloat16), ut_ref[...],
                               (((1,), (0,)), ((), ())),
                               preferred_element_type=jnp.float32)
        a = cumi - hfull
        cnt = cumi[:, _W - 1:_W]
        cm = jnp.minimum(cumi, 34.0)
        cols = [jnp.sum(jnp.where(cm <= np.float32(s), 1.0, 0.0),
                        axis=1, keepdims=True) for s in range(NSAMPLE)]
        beta_ref[...] = jnp.concatenate(cols, axis=1).astype(jnp.int32)
        zpad = jnp.zeros((_MT, 15), jnp.float32)
        a2_ref[...] = jnp.concatenate([a, cnt, zpad], axis=1)
        zpad2 = jnp.zeros((_MT, 16), jnp.float32)
        pw2_ref[...] = jnp.concatenate([pwfull, zpad2], axis=1).astype(jnp.int32)


def _ball_query(pT, np16, seg, p2m, ut):
    return pl.pallas_call(
        _ballq_body,
        grid=(B, M // _MT, _NCH),
        in_specs=[
            pl.BlockSpec((None, 3, _NC), lambda b, mt, nc: (b, 0, nc)),
            pl.BlockSpec((None, _MT, 16), lambda b, mt, nc: (b, mt, 0)),
            pl.BlockSpec((_NC, _WPC), lambda b, mt, nc: (0, 0)),
            pl.BlockSpec((_NC, _WPC), lambda b, mt, nc: (0, 0)),
            pl.BlockSpec((_W, _W), lambda b, mt, nc: (0, 0)),
        ],
        out_specs=(
            pl.BlockSpec((None, _MT, NSAMPLE), lambda b, mt, nc: (b, mt, 0)),
            pl.BlockSpec((None, _MT, _WP), lambda b, mt, nc: (b, mt, 0)),
            pl.BlockSpec((None, _MT, _WP), lambda b, mt, nc: (b, mt, 0)),
        ),
        out_shape=(jax.ShapeDtypeStruct((B, M, NSAMPLE), jnp.int32),
                   jax.ShapeDtypeStruct((B, M, _WP), jnp.float32),
                   jax.ShapeDtypeStruct((B, M, _WP), jnp.int32)),
        scratch_shapes=[pltpu.VMEM((_NCH, _MT, _WPC), jnp.float32),
                        pltpu.VMEM((_NCH, _MT, _WPC), jnp.float32)],
    )(pT, np16, seg, p2m, ut)


def _sc_ballq_fix(pw2, a2, beta):
    rows = beta.shape[0]
    per_w = rows // _NW
    grp = 16
    nb = per_w // grp
    mesh = plsc.VectorSubcoreMesh(core_axis_name="c", subcore_axis_name="s")

    @functools.partial(
        pl.kernel, mesh=mesh,
        out_type=jax.ShapeDtypeStruct((rows, NSAMPLE), jnp.int32),
        scratch_types=[pltpu.VMEM((grp, _WP), jnp.int32),
                       pltpu.VMEM((grp, _WP), jnp.float32),
                       pltpu.VMEM((grp, NSAMPLE), jnp.int32),
                       pltpu.VMEM((grp, NSAMPLE), jnp.int32)],
        compiler_params=pltpu.CompilerParams(use_tc_tiling_on_sc=False,
                                             needs_layout_passes=False),
    )
    def k(pw_hbm, a_hbm, b_hbm, out_hbm, pwv, av, bv, gv):
        wid = lax.axis_index("s") * 2 + lax.axis_index("c")
        iota16 = lax.broadcasted_iota(jnp.int32, (16,), 0)
        for b_i in range(nb):
            base = wid * per_w + b_i * grp
            pltpu.sync_copy(pw_hbm.at[pl.ds(base, grp)], pwv)
            pltpu.sync_copy(a_hbm.at[pl.ds(base, grp)], av)
            pltpu.sync_copy(b_hbm.at[pl.ds(base, grp)], bv)

            def row_body(r, carry):
                ridx = jnp.full((16,), r, jnp.int32)
                for g in range(2):
                    sidx = iota16 + g * 16
                    bvec = plsc.load_gather(bv, [ridx, sidx])
                    ab = plsc.load_gather(av, [ridx, bvec])
                    wb = plsc.load_gather(pwv, [ridx, bvec])
                    t = sidx.astype(jnp.float32) - ab
                    run = jnp.zeros((16,), jnp.int32)
                    corr = jnp.zeros((16,), jnp.float32)
                    for l in range(16):
                        run = run + (lax.shift_right_logical(wb, l) & 1)
                        corr = corr + jnp.where(
                            run.astype(jnp.float32) <= t, 1.0, 0.0)
                    gsl = bvec * 16 + corr.astype(jnp.int32)
                    plsc.store_scatter(gv, [ridx, sidx], gsl)
                cntv = plsc.load_gather(av, [ridx, jnp.full((16,), _W, jnp.int32)])
                firstv = plsc.load_gather(gv, [ridx, jnp.zeros((16,), jnp.int32)])
                firstv = jnp.where(cntv > 0, firstv, 0)
                for g in range(2):
                    sidx = iota16 + g * 16
                    cur = plsc.load_gather(gv, [ridx, sidx])
                    padded = jnp.where(sidx.astype(jnp.float32) < cntv,
                                       cur, firstv)
                    plsc.store_scatter(gv, [ridx, sidx], padded)
                return carry

            lax.fori_loop(0, grp, row_body, 0)
            pltpu.sync_copy(gv, out_hbm.at[pl.ds(base, grp)])

    return k(pw2, a2, beta)


_NW = 32


def _sc_gather_centroids(p16, iflat):
    rows = iflat.shape[0]
    per_w = rows // _NW
    mesh = plsc.VectorSubcoreMesh(core_axis_name="c", subcore_axis_name="s")

    @functools.partial(
        pl.kernel, mesh=mesh,
        out_type=jax.ShapeDtypeStruct((rows, 16), jnp.float32),
        scratch_types=[pltpu.VMEM((per_w,), jnp.int32),
                       pltpu.VMEM((per_w, 16), jnp.float32),
                       pltpu.SemaphoreType.DMA],
        compiler_params=pltpu.CompilerParams(use_tc_tiling_on_sc=False),
    )
    def k(tab_hbm, idx_hbm, out_hbm, idx_v, rows_v, sem):
        wid = lax.axis_index("s") * 2 + lax.axis_index("c")
        base = wid * per_w
        pltpu.sync_copy(idx_hbm.at[pl.ds(base, per_w)], idx_v)
        pltpu.async_copy(tab_hbm.at[idx_v], rows_v, sem).wait()
        pltpu.sync_copy(rows_v, out_hbm.at[pl.ds(base, per_w)])

    return k(p16, iflat)


def _sc_gather_groups(f1t, p16, iflat):
    rows = iflat.shape[0]
    per_w = rows // _NW
    chunk = 1024
    nch = per_w // chunk
    mesh = plsc.VectorSubcoreMesh(core_axis_name="c", subcore_axis_name="s")

    @functools.partial(
        pl.kernel, mesh=mesh,
        out_type=(jax.ShapeDtypeStruct((rows, OUT_C), jnp.float32),
                  jax.ShapeDtypeStruct((rows, 16), jnp.float32)),
        scratch_types=[pltpu.VMEM((chunk,), jnp.int32),
                       pltpu.VMEM((chunk, OUT_C), jnp.float32),
                       pltpu.VMEM((chunk, 16), jnp.float32),
                       pltpu.SemaphoreType.DMA,
                       pltpu.SemaphoreType.DMA],
        compiler_params=pltpu.CompilerParams(use_tc_tiling_on_sc=False),
    )
    def k(f_hbm, p_hbm, idx_hbm, fj_hbm, gx_hbm, idx_v, f_v, x_v, sem1, sem2):
        wid = lax.axis_index("s") * 2 + lax.axis_index("c")
        base = wid * per_w
        for j in range(nch):
            off = base + j * chunk
            pltpu.sync_copy(idx_hbm.at[pl.ds(off, chunk)], idx_v)
            cp1 = pltpu.async_copy(f_hbm.at[idx_v], f_v, sem1)
            cp2 = pltpu.async_copy(p_hbm.at[idx_v], x_v, sem2)
            cp1.wait()
            cp2.wait()
            pltpu.sync_copy(f_v, fj_hbm.at[pl.ds(off, chunk)])
            pltpu.sync_copy(x_v, gx_hbm.at[pl.ds(off, chunk)])

    return k(f1t, p16, iflat)


def kernel(p, f, pe, W1_0, g1_0, b1_0, W1_1, g1_1, b1_1, W2_0, g2_0, b2_0,
           W2_1, g2_1, b2_1, W2_2, g2_2, b2_2):
    del pe
    r2 = lambda v: v.reshape(1, -1)

    idx = jax.random.randint(jax.random.key(42), (B, M), 0, N)
    offs = (jnp.arange(B, dtype=jnp.int32) * N)[:, None]
    iflat_c = (idx.astype(jnp.int32) + offs).reshape(-1)

    p16 = jnp.zeros((B * N, 16), jnp.float32).at[:, 0:3].set(p.reshape(B * N, 3))
    fT2 = jnp.transpose(f, (0, 2, 1)).reshape(B * N, IN_C)
    pT = jnp.transpose(p, (0, 2, 1))

    h0, s0, q0 = _mm_stats(fT2, W1_0, 2048)
    h1, s1, q1 = _bn_mm_stats(h0, s0, q0, r2(g1_0), r2(b1_0), W1_1, 2048)
    f1t = _bn_relu(h1, s1, q1, r2(g1_1), r2(b1_1), 2048)

    np16_flat = _sc_gather_centroids(p16, iflat_c)
    np16 = np16_flat.reshape(B, M, 16)
    new_p = np16[:, :, 0:3]

    nn = np.arange(_NC)
    ww = np.arange(_W)
    seg = jnp.asarray(nn[:, None] // 16 == np.arange(_WPC)[None, :],
                      jnp.bfloat16)
    p2m = jnp.asarray((nn[:, None] // 16 == np.arange(_WPC)[None, :])
                      * (2.0 ** (nn % 16))[:, None], jnp.bfloat16)
    ut = jnp.asarray(ww[:, None] <= ww[None, :], jnp.bfloat16)
    beta, a2, pw2 = _ball_query(pT, np16, seg, p2m, ut)
    gidx_flat = _sc_ballq_fix(pw2.reshape(B * M, _WP),
                              a2.reshape(B * M, _WP),
                              beta.reshape(B * M, NSAMPLE))
    gidx = gidx_flat.reshape(B, M, NSAMPLE)

    iflat_g = (gidx + offs[:, :, None]).reshape(-1)
    fj_flat, gx_flat = _sc_gather_groups(f1t, p16, iflat_g)

    npb = jnp.broadcast_to(np16_flat[:, None, :], (B * M, NSAMPLE, 16))
    npb = npb.reshape(BMS, 16)
    h1p, t1, u1 = _dp_mm_stats(gx_flat, npb, W2_0, 4096)
    h2p, t2, u2 = _bn_mm_stats(h1p, t1, u1, r2(g2_0), r2(b2_0), W2_1, 4096)
    h3p, t3, u3 = _bn_mm_stats(h2p, t2, u2, r2(g2_1), r2(b2_1), W2_2, 4096)
    pe_flat, fout_flat = _pe_max(h3p, t3, u3, r2(g2_2), r2(b2_2), fj_flat, 4096)

    pe_out = pe_flat.reshape(B, M, NSAMPLE, OUT_C).transpose(0, 3, 1, 2)
    fout = fout_flat.reshape(B, M, OUT_C).transpose(0, 2, 1)
    return new_p, fout, pe_out


# speedup vs baseline: 17.7659x; 1.0444x over previous
import functools

import numpy as np
import jax
import jax.numpy as jnp
from jax import lax
from jax.experimental import pallas as pl
from jax.experimental.pallas import tpu as pltpu
from jax.experimental.pallas import tpu_sc as plsc

B, N, IN_C, OUT_C = 2, 8192, 32, 64
STRIDE, NSAMPLE = 4, 32
M = N // STRIDE
BMS = B * M * NSAMPLE
R2 = np.float32(0.1 ** 2)
EPS = np.float32(1e-5)


def _stats_update(y, s_ref, q_ref, step):
    @pl.when(step == 0)
    def _():
        s_ref[...] = jnp.zeros_like(s_ref)
        q_ref[...] = jnp.zeros_like(q_ref)

    ones = jnp.ones((1, y.shape[0]), jnp.float32)
    s_ref[...] += lax.dot_general(ones, y, (((1,), (0,)), ((), ())),
                                  preferred_element_type=jnp.float32)
    q_ref[...] += lax.dot_general(ones, y * y, (((1,), (0,)), ((), ())),
                                  preferred_element_type=jnp.float32)


def _affine(s_in, q_in, g_ref, b_ref, ntot):
    mean = s_in[...] * np.float32(1.0 / ntot)
    var = q_in[...] * np.float32(1.0 / ntot) - mean * mean
    a = g_ref[...] / jnp.sqrt(var + EPS)
    c = b_ref[...] - mean * a
    return a, c


def _mm_stats_body(x_ref, w_ref, out_ref, s_ref, q_ref):
    y = lax.dot_general(x_ref[...], w_ref[...], (((1,), (1,)), ((), ())),
                        preferred_element_type=jnp.float32)
    out_ref[...] = y
    _stats_update(y, s_ref, q_ref, pl.program_id(0))


def _bn_mm_stats_body(x_ref, s_in, q_in, g_ref, b_ref, w_ref,
                      out_ref, s_ref, q_ref, *, ntot):
    a, c = _affine(s_in, q_in, g_ref, b_ref, ntot)
    h = jnp.maximum(x_ref[...] * a + c, 0.0)
    y = lax.dot_general(h, w_ref[...], (((1,), (1,)), ((), ())),
                        preferred_element_type=jnp.float32)
    out_ref[...] = y
    _stats_update(y, s_ref, q_ref, pl.program_id(0))


def _bn_relu_body(x_ref, s_in, q_in, g_ref, b_ref, out_ref, *, ntot):
    a, c = _affine(s_in, q_in, g_ref, b_ref, ntot)
    out_ref[...] = jnp.maximum(x_ref[...] * a + c, 0.0)


def _dp_mm_stats_body(gx_ref, np_ref, w_ref, out_ref, s_ref, q_ref):
    dp = gx_ref[:, 0:3] - np_ref[:, 0:3]
    y = lax.dot_general(dp, w_ref[...], (((1,), (1,)), ((), ())),
                        preferred_element_type=jnp.float32)
    out_ref[...] = y
    _stats_update(y, s_ref, q_ref, pl.program_id(0))


def _pe_max_body(x_ref, s_in, q_in, g_ref, b_ref, fj_ref,
                 pe_ref, fout_ref, *, ntot):
    a, c = _affine(s_in, q_in, g_ref, b_ref, ntot)
    pe = jnp.maximum(x_ref[...] * a + c, 0.0)
    pe_ref[...] = pe
    tot = pe + fj_ref[...]
    rt = tot.shape[0]
    fout_ref[...] = jnp.max(tot.reshape(rt // NSAMPLE, NSAMPLE, tot.shape[1]),
                            axis=1)


def _row_spec(rt, cols):
    return pl.BlockSpec((rt, cols), lambda i: (i, 0))


def _fix_spec(rows, cols):
    return pl.BlockSpec((rows, cols), lambda i: (0, 0))


def _stats_shapes(cout):
    return (jax.ShapeDtypeStruct((1, cout), jnp.float32),
            jax.ShapeDtypeStruct((1, cout), jnp.float32))


def _mm_stats(x, w, rt):
    rows, cout = x.shape[0], w.shape[0]
    return pl.pallas_call(
        _mm_stats_body,
        grid=(rows // rt,),
        in_specs=[_row_spec(rt, x.shape[1]), _fix_spec(*w.shape)],
        out_specs=(_row_spec(rt, cout), _fix_spec(1, cout), _fix_spec(1, cout)),
        out_shape=(jax.ShapeDtypeStruct((rows, cout), jnp.float32),
                   *_stats_shapes(cout)),
    )(x, w)


def _bn_mm_stats(x, s, q, g, b, w, rt):
    rows, cin, cout = x.shape[0], x.shape[1], w.shape[0]
    return pl.pallas_call(
        functools.partial(_bn_mm_stats_body, ntot=rows),
        grid=(rows // rt,),
        in_specs=[_row_spec(rt, cin), _fix_spec(1, cin), _fix_spec(1, cin),
                  _fix_spec(1, cin), _fix_spec(1, cin), _fix_spec(*w.shape)],
        out_specs=(_row_spec(rt, cout), _fix_spec(1, cout), _fix_spec(1, cout)),
        out_shape=(jax.ShapeDtypeStruct((rows, cout), jnp.float32),
                   *_stats_shapes(cout)),
    )(x, s, q, g, b, w)


def _bn_relu(x, s, q, g, b, rt):
    rows, cin = x.shape
    return pl.pallas_call(
        functools.partial(_bn_relu_body, ntot=rows),
        grid=(rows // rt,),
        in_specs=[_row_spec(rt, cin), _fix_spec(1, cin), _fix_spec(1, cin),
                  _fix_spec(1, cin), _fix_spec(1, cin)],
        out_specs=_row_spec(rt, cin),
        out_shape=jax.ShapeDtypeStruct((rows, cin), jnp.float32),
    )(x, s, q, g, b)


def _dp_mm_stats(gxyz, npb, w, rt):
    rows, cout = gxyz.shape[0], w.shape[0]
    return pl.pallas_call(
        _dp_mm_stats_body,
        grid=(rows // rt,),
        in_specs=[_row_spec(rt, 16), _row_spec(rt, 16), _fix_spec(*w.shape)],
        out_specs=(_row_spec(rt, cout), _fix_spec(1, cout), _fix_spec(1, cout)),
        out_shape=(jax.ShapeDtypeStruct((rows, cout), jnp.float32),
                   *_stats_shapes(cout)),
    )(gxyz, npb, w)


def _pe_max(x, s, q, g, b, fj, rt):
    rows, cin = x.shape
    return pl.pallas_call(
        functools.partial(_pe_max_body, ntot=rows),
        grid=(rows // rt,),
        in_specs=[_row_spec(rt, cin), _fix_spec(1, cin), _fix_spec(1, cin),
                  _fix_spec(1, cin), _fix_spec(1, cin), _row_spec(rt, cin)],
        out_specs=(_row_spec(rt, cin),
                   pl.BlockSpec((rt // NSAMPLE, cin), lambda i: (i, 0))),
        out_shape=(jax.ShapeDtypeStruct((rows, cin), jnp.float32),
                   jax.ShapeDtypeStruct((rows // NSAMPLE, cin), jnp.float32)),
    )(x, s, q, g, b, fj)


_MT = 256
_NC = 512
_NCH = N // _NC
_WPC = _NC // 16
_W = N // 16
_WP = _W + 16


def _ballq_body(pT_ref, np_ref, seg_ref, p2m_ref, ut_ref,
                beta_ref, a2_ref, pw2_ref, hscr, pwscr):
    nc = pl.program_id(2)
    qx = np_ref[:, 0:1]
    qy = np_ref[:, 1:2]
    qz = np_ref[:, 2:3]
    px = pT_ref[0:1, :]
    py = pT_ref[1:2, :]
    pz = pT_ref[2:3, :]
    dx = qx - px
    dy = qy - py
    dz = qz - pz
    d2 = (dx * dx + dy * dy) + dz * dz
    wf = jnp.where(d2 < R2, 1.0, 0.0).astype(jnp.bfloat16)

    hscr[nc] = lax.dot_general(wf, seg_ref[...], (((1,), (0,)), ((), ())),
                               preferred_element_type=jnp.float32)
    pwscr[nc] = lax.dot_general(wf, p2m_ref[...], (((1,), (0,)), ((), ())),
                                preferred_element_type=jnp.float32)

    @pl.when(nc == _NCH - 1)
    def _():
        hfull = jnp.concatenate([hscr[k] for k in range(_NCH)], axis=1)
        pwfull = jnp.concatenate([pwscr[k] for k in range(_NCH)], axis=1)
        cumi = lax.dot_general(hfull.astype(jnp.bfloat16), ut_ref[...],
                               (((1,), (0,)), ((), ())),
                               preferred_element_type=jnp.float32)
        a = cumi - hfull
        cnt = cumi[:, _W - 1:_W]
        cm = jnp.minimum(cumi, 34.0)
        cols = [jnp.sum(jnp.where(cm <= np.float32(s), 1.0, 0.0),
                        axis=1, keepdims=True) for s in range(NSAMPLE)]
        beta_ref[...] = jnp.concatenate(cols, axis=1).astype(jnp.int32)
        zpad = jnp.zeros((_MT, 15), jnp.float32)
        a2_ref[...] = jnp.concatenate([a, cnt, zpad], axis=1)
        zpad2 = jnp.zeros((_MT, 16), jnp.float32)
        pw2_ref[...] = jnp.concatenate([pwfull, zpad2], axis=1).astype(jnp.int32)


def _ball_query(pT, np16, seg, p2m, ut):
    return pl.pallas_call(
        _ballq_body,
        grid=(B, M // _MT, _NCH),
        in_specs=[
            pl.BlockSpec((None, 3, _NC), lambda b, mt, nc: (b, 0, nc)),
            pl.BlockSpec((None, _MT, 16), lambda b, mt, nc: (b, mt, 0)),
            pl.BlockSpec((_NC, _WPC), lambda b, mt, nc: (0, 0)),
            pl.BlockSpec((_NC, _WPC), lambda b, mt, nc: (0, 0)),
            pl.BlockSpec((_W, _W), lambda b, mt, nc: (0, 0)),
        ],
        out_specs=(
            pl.BlockSpec((None, _MT, NSAMPLE), lambda b, mt, nc: (b, mt, 0)),
            pl.BlockSpec((None, _MT, _WP), lambda b, mt, nc: (b, mt, 0)),
            pl.BlockSpec((None, _MT, _WP), lambda b, mt, nc: (b, mt, 0)),
        ),
        out_shape=(jax.ShapeDtypeStruct((B, M, NSAMPLE), jnp.int32),
                   jax.ShapeDtypeStruct((B, M, _WP), jnp.float32),
                   jax.ShapeDtypeStruct((B, M, _WP), jnp.int32)),
        scratch_shapes=[pltpu.VMEM((_NCH, _MT, _WPC), jnp.float32),
                        pltpu.VMEM((_NCH, _MT, _WPC), jnp.float32)],
    )(pT, np16, seg, p2m, ut)


def _sc_ballq_fix(pw2, a2, beta):
    rows = beta.shape[0]
    per_w = rows // _NW
    grp = 16
    nb = per_w // grp
    mesh = plsc.VectorSubcoreMesh(core_axis_name="c", subcore_axis_name="s")

    @functools.partial(
        pl.kernel, mesh=mesh,
        out_type=jax.ShapeDtypeStruct((rows, NSAMPLE), jnp.int32),
        scratch_types=[pltpu.VMEM((grp, _WP), jnp.int32),
                       pltpu.VMEM((grp, _WP), jnp.float32),
                       pltpu.VMEM((grp, NSAMPLE), jnp.int32),
                       pltpu.VMEM((grp, NSAMPLE), jnp.int32)],
        compiler_params=pltpu.CompilerParams(use_tc_tiling_on_sc=False,
                                             needs_layout_passes=False),
    )
    def k(pw_hbm, a_hbm, b_hbm, out_hbm, pwv, av, bv, gv):
        wid = lax.axis_index("s") * 2 + lax.axis_index("c")
        iota16 = lax.broadcasted_iota(jnp.int32, (16,), 0)
        for b_i in range(nb):
            base = wid * per_w + b_i * grp
            pltpu.sync_copy(pw_hbm.at[pl.ds(base, grp)], pwv)
            pltpu.sync_copy(a_hbm.at[pl.ds(base, grp)], av)
            pltpu.sync_copy(b_hbm.at[pl.ds(base, grp)], bv)

            def row_body(r, carry):
                ridx = jnp.full((16,), r, jnp.int32)
                for g in range(2):
                    sidx = iota16 + g * 16
                    bvec = plsc.load_gather(bv, [ridx, sidx])
                    ab = plsc.load_gather(av, [ridx, bvec])
                    wb = plsc.load_gather(pwv, [ridx, bvec])
                    t = sidx.astype(jnp.float32) - ab
                    run = jnp.zeros((16,), jnp.int32)
                    corr = jnp.zeros((16,), jnp.float32)
                    for l in range(16):
                        run = run + (lax.shift_right_logical(wb, l) & 1)
                        corr = corr + jnp.where(
                            run.astype(jnp.float32) <= t, 1.0, 0.0)
                    gsl = bvec * 16 + corr.astype(jnp.int32)
                    plsc.store_scatter(gv, [ridx, sidx], gsl)
                cntv = plsc.load_gather(av, [ridx, jnp.full((16,), _W, jnp.int32)])
                firstv = plsc.load_gather(gv, [ridx, jnp.zeros((16,), jnp.int32)])
                firstv = jnp.where(cntv > 0, firstv, 0)
                for g in range(2):
                    sidx = iota16 + g * 16
                    cur = plsc.load_gather(gv, [ridx, sidx])
                    padded = jnp.where(sidx.astype(jnp.float32) < cntv,
                                       cur, firstv)
                    plsc.store_scatter(gv, [ridx, sidx], padded)
                return carry

            lax.fori_loop(0, grp, row_body, 0)
            pltpu.sync_copy(gv, out_hbm.at[pl.ds(base, grp)])

    return k(pw2, a2, beta)


_NW = 32


def _sc_gather_centroids(p16, iflat):
    rows = iflat.shape[0]
    per_w = rows // _NW
    mesh = plsc.VectorSubcoreMesh(core_axis_name="c", subcore_axis_name="s")

    @functools.partial(
        pl.kernel, mesh=mesh,
        out_type=jax.ShapeDtypeStruct((rows, 16), jnp.float32),
        scratch_types=[pltpu.VMEM((per_w,), jnp.int32),
                       pltpu.VMEM((per_w, 16), jnp.float32),
                       pltpu.SemaphoreType.DMA],
        compiler_params=pltpu.CompilerParams(use_tc_tiling_on_sc=False),
    )
    def k(tab_hbm, idx_hbm, out_hbm, idx_v, rows_v, sem):
        wid = lax.axis_index("s") * 2 + lax.axis_index("c")
        base = wid * per_w
        pltpu.sync_copy(idx_hbm.at[pl.ds(base, per_w)], idx_v)
        pltpu.async_copy(tab_hbm.at[idx_v], rows_v, sem).wait()
        pltpu.sync_copy(rows_v, out_hbm.at[pl.ds(base, per_w)])

    return k(p16, iflat)


def _sc_gather_groups(f1t, p16, iflat):
    rows = iflat.shape[0]
    per_w = rows // _NW
    chunk = 1024
    nch = per_w // chunk
    mesh = plsc.VectorSubcoreMesh(core_axis_name="c", subcore_axis_name="s")

    @functools.partial(
        pl.kernel, mesh=mesh,
        out_type=(jax.ShapeDtypeStruct((rows, OUT_C), jnp.float32),
                  jax.ShapeDtypeStruct((rows, 16), jnp.float32)),
        scratch_types=[pltpu.VMEM((chunk,), jnp.int32),
                       pltpu.VMEM((chunk, OUT_C), jnp.float32),
                       pltpu.VMEM((chunk, 16), jnp.float32),
                       pltpu.SemaphoreType.DMA,
                       pltpu.SemaphoreType.DMA],
        compiler_params=pltpu.CompilerParams(use_tc_tiling_on_sc=False),
    )
    def k(f_hbm, p_hbm, idx_hbm, fj_hbm, gx_hbm, idx_v, f_v, x_v, sem1, sem2):
        wid = lax.axis_index("s") * 2 + lax.axis_index("c")
        base = wid * per_w
        for j in range(nch):
            off = base + j * chunk
            pltpu.sync_copy(idx_hbm.at[pl.ds(off, chunk)], idx_v)
            cp1 = pltpu.async_copy(f_hbm.at[idx_v], f_v, sem1)
            cp2 = pltpu.async_copy(p_hbm.at[idx_v], x_v, sem2)
            cp1.wait()
            cp2.wait()
            pltpu.sync_copy(f_v, fj_hbm.at[pl.ds(off, chunk)])
            pltpu.sync_copy(x_v, gx_hbm.at[pl.ds(off, chunk)])

    return k(f1t, p16, iflat)


def kernel(p, f, pe, W1_0, g1_0, b1_0, W1_1, g1_1, b1_1, W2_0, g2_0, b2_0,
           W2_1, g2_1, b2_1, W2_2, g2_2, b2_2):
    del pe
    r2 = lambda v: v.reshape(1, -1)

    idx = jax.random.randint(jax.random.key(42), (B, M), 0, N)
    offs = (jnp.arange(B, dtype=jnp.int32) * N)[:, None]
    iflat_c = (idx.astype(jnp.int32) + offs).reshape(-1)

    p16 = jnp.zeros((B * N, 16), jnp.float32).at[:, 0:3].set(p.reshape(B * N, 3))
    fT2 = jnp.transpose(f, (0, 2, 1)).reshape(B * N, IN_C)
    pT = jnp.transpose(p, (0, 2, 1))

    h0, s0, q0 = _mm_stats(fT2, W1_0, 2048)
    h1, s1, q1 = _bn_mm_stats(h0, s0, q0, r2(g1_0), r2(b1_0), W1_1, 2048)
    f1t = _bn_relu(h1, s1, q1, r2(g1_1), r2(b1_1), 2048)

    np16_flat = _sc_gather_centroids(p16, iflat_c)
    np16 = np16_flat.reshape(B, M, 16)
    new_p = np16[:, :, 0:3]

    nn = np.arange(_NC)
    ww = np.arange(_W)
    seg = jnp.asarray(nn[:, None] // 16 == np.arange(_WPC)[None, :],
                      jnp.bfloat16)
    p2m = jnp.asarray((nn[:, None] // 16 == np.arange(_WPC)[None, :])
                      * (2.0 ** (nn % 16))[:, None], jnp.bfloat16)
    ut = jnp.asarray(ww[:, None] <= ww[None, :], jnp.bfloat16)
    beta, a2, pw2 = _ball_query(pT, np16, seg, p2m, ut)
    gidx_flat = _sc_ballq_fix(pw2.reshape(B * M, _WP),
                              a2.reshape(B * M, _WP),
                              beta.reshape(B * M, NSAMPLE))
    gidx = gidx_flat.reshape(B, M, NSAMPLE)

    iflat_g = (gidx + offs[:, :, None]).reshape(-1)
    fj_flat, gx_flat = _sc_gather_groups(f1t, p16, iflat_g)

    npb = jnp.broadcast_to(np16_flat[:, None, :], (B * M, NSAMPLE, 16))
    npb = npb.reshape(BMS, 16)
    h1p, t1, u1 = _dp_mm_stats(gx_flat, npb, W2_0, 8192)
    h2p, t2, u2 = _bn_mm_stats(h1p, t1, u1, r2(g2_0), r2(b2_0), W2_1, 8192)
    h3p, t3, u3 = _bn_mm_stats(h2p, t2, u2, r2(g2_1), r2(b2_1), W2_2, 8192)
    pe_flat, fout_flat = _pe_max(h3p, t3, u3, r2(g2_2), r2(b2_2), fj_flat, 8192)

    pe_out = pe_flat.reshape(B, M, NSAMPLE, OUT_C).transpose(0, 3, 1, 2)
    fout = fout_flat.reshape(B, M, OUT_C).transpose(0, 2, 1)
    return new_p, fout, pe_out


# speedup vs baseline: 18.4261x; 1.0372x over previous
import functools

import numpy as np
import jax
import jax.numpy as jnp
from jax import lax
from jax.experimental import pallas as pl
from jax.experimental.pallas import tpu as pltpu
from jax.experimental.pallas import tpu_sc as plsc

B, N, IN_C, OUT_C = 2, 8192, 32, 64
STRIDE, NSAMPLE = 4, 32
M = N // STRIDE
BMS = B * M * NSAMPLE
R2 = np.float32(0.1 ** 2)
EPS = np.float32(1e-5)


def _stats_update(y, s_ref, q_ref, step):
    @pl.when(step == 0)
    def _():
        s_ref[...] = jnp.zeros_like(s_ref)
        q_ref[...] = jnp.zeros_like(q_ref)

    ones = jnp.ones((1, y.shape[0]), jnp.float32)
    s_ref[...] += lax.dot_general(ones, y, (((1,), (0,)), ((), ())),
                                  preferred_element_type=jnp.float32)
    q_ref[...] += lax.dot_general(ones, y * y, (((1,), (0,)), ((), ())),
                                  preferred_element_type=jnp.float32)


def _affine(s_in, q_in, g_ref, b_ref, ntot):
    mean = s_in[...] * np.float32(1.0 / ntot)
    var = q_in[...] * np.float32(1.0 / ntot) - mean * mean
    a = g_ref[...] / jnp.sqrt(var + EPS)
    c = b_ref[...] - mean * a
    return a, c


def _mm_stats_body(x_ref, w_ref, out_ref, s_ref, q_ref):
    y = lax.dot_general(x_ref[...], w_ref[...], (((1,), (1,)), ((), ())),
                        preferred_element_type=jnp.float32)
    out_ref[...] = y
    _stats_update(y, s_ref, q_ref, pl.program_id(0))


def _bn_mm_stats_body(x_ref, s_in, q_in, g_ref, b_ref, w_ref,
                      out_ref, s_ref, q_ref, *, ntot):
    a, c = _affine(s_in, q_in, g_ref, b_ref, ntot)
    h = jnp.maximum(x_ref[...] * a + c, 0.0)
    y = lax.dot_general(h, w_ref[...], (((1,), (1,)), ((), ())),
                        preferred_element_type=jnp.float32)
    out_ref[...] = y
    _stats_update(y, s_ref, q_ref, pl.program_id(0))


def _bn_relu_body(x_ref, s_in, q_in, g_ref, b_ref, out_ref, *, ntot):
    a, c = _affine(s_in, q_in, g_ref, b_ref, ntot)
    out_ref[...] = jnp.maximum(x_ref[...] * a + c, 0.0)


def _stats_only_body(x_ref, s_in, q_in, g_ref, b_ref, w_ref, s_ref, q_ref,
                     *, ntot):
    a, c = _affine(s_in, q_in, g_ref, b_ref, ntot)
    h = jnp.maximum(x_ref[...] * a + c, 0.0)
    y = lax.dot_general(h, w_ref[...], (((1,), (1,)), ((), ())),
                        preferred_element_type=jnp.float32)
    _stats_update(y, s_ref, q_ref, pl.program_id(0))


def _bn2_mm_stats_body(x_ref, s0_in, q0_in, g0_ref, b0_ref, w0_ref,
                       s1_in, q1_in, g1_ref, b1_ref, w1_ref,
                       out_ref, s_ref, q_ref, *, ntot):
    a0, c0 = _affine(s0_in, q0_in, g0_ref, b0_ref, ntot)
    h0 = jnp.maximum(x_ref[...] * a0 + c0, 0.0)
    y0 = lax.dot_general(h0, w0_ref[...], (((1,), (1,)), ((), ())),
                         preferred_element_type=jnp.float32)
    a1, c1 = _affine(s1_in, q1_in, g1_ref, b1_ref, ntot)
    h1 = jnp.maximum(y0 * a1 + c1, 0.0)
    y = lax.dot_general(h1, w1_ref[...], (((1,), (1,)), ((), ())),
                        preferred_element_type=jnp.float32)
    out_ref[...] = y
    _stats_update(y, s_ref, q_ref, pl.program_id(0))


def _dp_mm_stats_body(gx_ref, np_ref, w_ref, out_ref, s_ref, q_ref):
    rows = gx_ref.shape[0]
    npb = jnp.broadcast_to(np_ref[...][:, None, :],
                           (rows // NSAMPLE, NSAMPLE, 16)).reshape(rows, 16)
    dp = gx_ref[:, 0:3] - npb[:, 0:3]
    y = lax.dot_general(dp, w_ref[...], (((1,), (1,)), ((), ())),
                        preferred_element_type=jnp.float32)
    out_ref[...] = y
    _stats_update(y, s_ref, q_ref, pl.program_id(0))


def _pe_max_body(x_ref, s_in, q_in, g_ref, b_ref, fj_ref,
                 pe_ref, fout_ref, *, ntot):
    a, c = _affine(s_in, q_in, g_ref, b_ref, ntot)
    pe = jnp.maximum(x_ref[...] * a + c, 0.0)
    pe_ref[...] = pe
    tot = pe + fj_ref[...]
    rt = tot.shape[0]
    fout_ref[...] = jnp.max(tot.reshape(rt // NSAMPLE, NSAMPLE, tot.shape[1]),
                            axis=1)


def _row_spec(rt, cols):
    return pl.BlockSpec((rt, cols), lambda i: (i, 0))


def _fix_spec(rows, cols):
    return pl.BlockSpec((rows, cols), lambda i: (0, 0))


def _stats_shapes(cout):
    return (jax.ShapeDtypeStruct((1, cout), jnp.float32),
            jax.ShapeDtypeStruct((1, cout), jnp.float32))


def _mm_stats(x, w, rt):
    rows, cout = x.shape[0], w.shape[0]
    return pl.pallas_call(
        _mm_stats_body,
        grid=(rows // rt,),
        in_specs=[_row_spec(rt, x.shape[1]), _fix_spec(*w.shape)],
        out_specs=(_row_spec(rt, cout), _fix_spec(1, cout), _fix_spec(1, cout)),
        out_shape=(jax.ShapeDtypeStruct((rows, cout), jnp.float32),
                   *_stats_shapes(cout)),
    )(x, w)


def _bn_mm_stats(x, s, q, g, b, w, rt):
    rows, cin, cout = x.shape[0], x.shape[1], w.shape[0]
    return pl.pallas_call(
        functools.partial(_bn_mm_stats_body, ntot=rows),
        grid=(rows // rt,),
        in_specs=[_row_spec(rt, cin), _fix_spec(1, cin), _fix_spec(1, cin),
                  _fix_spec(1, cin), _fix_spec(1, cin), _fix_spec(*w.shape)],
        out_specs=(_row_spec(rt, cout), _fix_spec(1, cout), _fix_spec(1, cout)),
        out_shape=(jax.ShapeDtypeStruct((rows, cout), jnp.float32),
                   *_stats_shapes(cout)),
    )(x, s, q, g, b, w)


def _bn_relu(x, s, q, g, b, rt):
    rows, cin = x.shape
    return pl.pallas_call(
        functools.partial(_bn_relu_body, ntot=rows),
        grid=(rows // rt,),
        in_specs=[_row_spec(rt, cin), _fix_spec(1, cin), _fix_spec(1, cin),
                  _fix_spec(1, cin), _fix_spec(1, cin)],
        out_specs=_row_spec(rt, cin),
        out_shape=jax.ShapeDtypeStruct((rows, cin), jnp.float32),
    )(x, s, q, g, b)


def _dp_mm_stats(gxyz, np16f, w, rt):
    rows, cout = gxyz.shape[0], w.shape[0]
    return pl.pallas_call(
        _dp_mm_stats_body,
        grid=(rows // rt,),
        in_specs=[_row_spec(rt, 16), _row_spec(rt // NSAMPLE, 16),
                  _fix_spec(*w.shape)],
        out_specs=(_row_spec(rt, cout), _fix_spec(1, cout), _fix_spec(1, cout)),
        out_shape=(jax.ShapeDtypeStruct((rows, cout), jnp.float32),
                   *_stats_shapes(cout)),
    )(gxyz, np16f, w)


def _stats_only(x, s, q, g, b, w, rt):
    rows, cin, cout = x.shape[0], x.shape[1], w.shape[0]
    return pl.pallas_call(
        functools.partial(_stats_only_body, ntot=rows),
        grid=(rows // rt,),
        in_specs=[_row_spec(rt, cin), _fix_spec(1, cin), _fix_spec(1, cin),
                  _fix_spec(1, cin), _fix_spec(1, cin), _fix_spec(*w.shape)],
        out_specs=(_fix_spec(1, cout), _fix_spec(1, cout)),
        out_shape=_stats_shapes(cout),
    )(x, s, q, g, b, w)


def _bn2_mm_stats(x, s0, q0, g0, b0, w0, s1, q1, g1, b1, w1, rt):
    rows, cin = x.shape
    cmid, cout = w0.shape[0], w1.shape[0]
    return pl.pallas_call(
        functools.partial(_bn2_mm_stats_body, ntot=rows),
        grid=(rows // rt,),
        in_specs=[_row_spec(rt, cin), _fix_spec(1, cin), _fix_spec(1, cin),
                  _fix_spec(1, cin), _fix_spec(1, cin), _fix_spec(*w0.shape),
                  _fix_spec(1, cmid), _fix_spec(1, cmid),
                  _fix_spec(1, cmid), _fix_spec(1, cmid), _fix_spec(*w1.shape)],
        out_specs=(_row_spec(rt, cout), _fix_spec(1, cout), _fix_spec(1, cout)),
        out_shape=(jax.ShapeDtypeStruct((rows, cout), jnp.float32),
                   *_stats_shapes(cout)),
    )(x, s0, q0, g0, b0, w0, s1, q1, g1, b1, w1)


def _pe_max(x, s, q, g, b, fj, rt):
    rows, cin = x.shape
    return pl.pallas_call(
        functools.partial(_pe_max_body, ntot=rows),
        grid=(rows // rt,),
        in_specs=[_row_spec(rt, cin), _fix_spec(1, cin), _fix_spec(1, cin),
                  _fix_spec(1, cin), _fix_spec(1, cin), _row_spec(rt, cin)],
        out_specs=(_row_spec(rt, cin),
                   pl.BlockSpec((rt // NSAMPLE, cin), lambda i: (i, 0))),
        out_shape=(jax.ShapeDtypeStruct((rows, cin), jnp.float32),
                   jax.ShapeDtypeStruct((rows // NSAMPLE, cin), jnp.float32)),
    )(x, s, q, g, b, fj)


_MT = 256
_NC = 512
_NCH = N // _NC
_WPC = _NC // 16
_W = N // 16
_WP = _W + 16


def _ballq_body(pT_ref, np_ref, seg_ref, p2m_ref, ut_ref,
                beta_ref, a2_ref, pw2_ref, hscr, pwscr):
    nc = pl.program_id(2)
    qx = np_ref[:, 0:1]
    qy = np_ref[:, 1:2]
    qz = np_ref[:, 2:3]
    px = pT_ref[0:1, :]
    py = pT_ref[1:2, :]
    pz = pT_ref[2:3, :]
    dx = qx - px
    dy = qy - py
    dz = qz - pz
    d2 = (dx * dx + dy * dy) + dz * dz
    wf = jnp.where(d2 < R2, 1.0, 0.0).astype(jnp.bfloat16)

    hscr[nc] = lax.dot_general(wf, seg_ref[...], (((1,), (0,)), ((), ())),
                               preferred_element_type=jnp.float32)
    pwscr[nc] = lax.dot_general(wf, p2m_ref[...], (((1,), (0,)), ((), ())),
                                preferred_element_type=jnp.float32)

    @pl.when(nc == _NCH - 1)
    def _():
        hfull = jnp.concatenate([hscr[k] for k in range(_NCH)], axis=1)
        pwfull = jnp.concatenate([pwscr[k] for k in range(_NCH)], axis=1)
        cumi = lax.dot_general(hfull.astype(jnp.bfloat16), ut_ref[...],
                               (((1,), (0,)), ((), ())),
                               preferred_element_type=jnp.float32)
        a = cumi - hfull
        cnt = cumi[:, _W - 1:_W]
        cm = jnp.minimum(cumi, 34.0)
        cols = [jnp.sum(jnp.where(cm <= np.float32(s), 1.0, 0.0),
                        axis=1, keepdims=True) for s in range(NSAMPLE)]
        beta_ref[...] = jnp.concatenate(cols, axis=1).astype(jnp.int32)
        zpad = jnp.zeros((_MT, 15), jnp.float32)
        a2_ref[...] = jnp.concatenate([a, cnt, zpad], axis=1)
        zpad2 = jnp.zeros((_MT, 16), jnp.float32)
        pw2_ref[...] = jnp.concatenate([pwfull, zpad2], axis=1).astype(jnp.int32)


def _ball_query(pT, np16, seg, p2m, ut):
    return pl.pallas_call(
        _ballq_body,
        grid=(B, M // _MT, _NCH),
        in_specs=[
            pl.BlockSpec((None, 3, _NC), lambda b, mt, nc: (b, 0, nc)),
            pl.BlockSpec((None, _MT, 16), lambda b, mt, nc: (b, mt, 0)),
            pl.BlockSpec((_NC, _WPC), lambda b, mt, nc: (0, 0)),
            pl.BlockSpec((_NC, _WPC), lambda b, mt, nc: (0, 0)),
            pl.BlockSpec((_W, _W), lambda b, mt, nc: (0, 0)),
        ],
        out_specs=(
            pl.BlockSpec((None, _MT, NSAMPLE), lambda b, mt, nc: (b, mt, 0)),
            pl.BlockSpec((None, _MT, _WP), lambda b, mt, nc: (b, mt, 0)),
            pl.BlockSpec((None, _MT, _WP), lambda b, mt, nc: (b, mt, 0)),
        ),
        out_shape=(jax.ShapeDtypeStruct((B, M, NSAMPLE), jnp.int32),
                   jax.ShapeDtypeStruct((B, M, _WP), jnp.float32),
                   jax.ShapeDtypeStruct((B, M, _WP), jnp.int32)),
        scratch_shapes=[pltpu.VMEM((_NCH, _MT, _WPC), jnp.float32),
                        pltpu.VMEM((_NCH, _MT, _WPC), jnp.float32)],
    )(pT, np16, seg, p2m, ut)


def _sc_ballq_fix(pw2, a2, beta):
    rows = beta.shape[0]
    per_w = rows // _NW
    grp = 16
    nb = per_w // grp
    mesh = plsc.VectorSubcoreMesh(core_axis_name="c", subcore_axis_name="s")

    @functools.partial(
        pl.kernel, mesh=mesh,
        out_type=jax.ShapeDtypeStruct((rows, NSAMPLE), jnp.int32),
        scratch_types=[pltpu.VMEM((grp, _WP), jnp.int32),
                       pltpu.VMEM((grp, _WP), jnp.float32),
                       pltpu.VMEM((grp, NSAMPLE), jnp.int32),
                       pltpu.VMEM((grp, NSAMPLE), jnp.int32)],
        compiler_params=pltpu.CompilerParams(use_tc_tiling_on_sc=False,
                                             needs_layout_passes=False),
    )
    def k(pw_hbm, a_hbm, b_hbm, out_hbm, pwv, av, bv, gv):
        wid = lax.axis_index("s") * 2 + lax.axis_index("c")
        iota16 = lax.broadcasted_iota(jnp.int32, (16,), 0)
        for b_i in range(nb):
            base = wid * per_w + b_i * grp
            pltpu.sync_copy(pw_hbm.at[pl.ds(base, grp)], pwv)
            pltpu.sync_copy(a_hbm.at[pl.ds(base, grp)], av)
            pltpu.sync_copy(b_hbm.at[pl.ds(base, grp)], bv)

            def row_body(r, carry):
                ridx = jnp.full((16,), r, jnp.int32)
                for g in range(2):
                    sidx = iota16 + g * 16
                    bvec = plsc.load_gather(bv, [ridx, sidx])
                    ab = plsc.load_gather(av, [ridx, bvec])
                    wb = plsc.load_gather(pwv, [ridx, bvec])
                    t = sidx.astype(jnp.float32) - ab
                    run = jnp.zeros((16,), jnp.int32)
                    corr = jnp.zeros((16,), jnp.float32)
                    for l in range(16):
                        run = run + (lax.shift_right_logical(wb, l) & 1)
                        corr = corr + jnp.where(
                            run.astype(jnp.float32) <= t, 1.0, 0.0)
                    gsl = bvec * 16 + corr.astype(jnp.int32)
                    plsc.store_scatter(gv, [ridx, sidx], gsl)
                cntv = plsc.load_gather(av, [ridx, jnp.full((16,), _W, jnp.int32)])
                firstv = plsc.load_gather(gv, [ridx, jnp.zeros((16,), jnp.int32)])
                firstv = jnp.where(cntv > 0, firstv, 0)
                for g in range(2):
                    sidx = iota16 + g * 16
                    cur = plsc.load_gather(gv, [ridx, sidx])
                    padded = jnp.where(sidx.astype(jnp.float32) < cntv,
                                       cur, firstv)
                    plsc.store_scatter(gv, [ridx, sidx], padded)
                return carry

            lax.fori_loop(0, grp, row_body, 0)
            pltpu.sync_copy(gv, out_hbm.at[pl.ds(base, grp)])

    return k(pw2, a2, beta)


_NW = 32


def _sc_gather_rows(tab, iflat, width, chunk):
    rows = iflat.shape[0]
    per_w = rows // _NW
    nch = per_w // chunk
    mesh = plsc.VectorSubcoreMesh(core_axis_name="c", subcore_axis_name="s")

    @functools.partial(
        pl.kernel, mesh=mesh,
        out_type=jax.ShapeDtypeStruct((rows, width), jnp.float32),
        scratch_types=[pltpu.VMEM((chunk,), jnp.int32),
                       pltpu.VMEM((chunk, width), jnp.float32),
                       pltpu.SemaphoreType.DMA],
        compiler_params=pltpu.CompilerParams(use_tc_tiling_on_sc=False),
    )
    def k(tab_hbm, idx_hbm, out_hbm, idx_v, row_v, sem):
        wid = lax.axis_index("s") * 2 + lax.axis_index("c")
        base = wid * per_w
        for j in range(nch):
            off = base + j * chunk
            pltpu.sync_copy(idx_hbm.at[pl.ds(off, chunk)], idx_v)
            pltpu.async_copy(tab_hbm.at[idx_v], row_v, sem).wait()
            pltpu.sync_copy(row_v, out_hbm.at[pl.ds(off, chunk)])

    return k(tab, iflat)


def kernel(p, f, pe, W1_0, g1_0, b1_0, W1_1, g1_1, b1_1, W2_0, g2_0, b2_0,
           W2_1, g2_1, b2_1, W2_2, g2_2, b2_2):
    del pe
    r2 = lambda v: v.reshape(1, -1)

    idx = jax.random.randint(jax.random.key(42), (B, M), 0, N)
    offs = (jnp.arange(B, dtype=jnp.int32) * N)[:, None]
    iflat_c = (idx.astype(jnp.int32) + offs).reshape(-1)

    p16 = jnp.zeros((B * N, 16), jnp.float32).at[:, 0:3].set(p.reshape(B * N, 3))
    fT2 = jnp.transpose(f, (0, 2, 1)).reshape(B * N, IN_C)
    pT = jnp.transpose(p, (0, 2, 1))

    h0, s0, q0 = _mm_stats(fT2, W1_0, 2048)
    h1, s1, q1 = _bn_mm_stats(h0, s0, q0, r2(g1_0), r2(b1_0), W1_1, 2048)
    f1t = _bn_relu(h1, s1, q1, r2(g1_1), r2(b1_1), 2048)

    np16_flat = _sc_gather_rows(p16, iflat_c, 16, 128)
    np16 = np16_flat.reshape(B, M, 16)
    new_p = np16[:, :, 0:3]

    nn = np.arange(_NC)
    ww = np.arange(_W)
    seg = jnp.asarray(nn[:, None] // 16 == np.arange(_WPC)[None, :],
                      jnp.bfloat16)
    p2m = jnp.asarray((nn[:, None] // 16 == np.arange(_WPC)[None, :])
                      * (2.0 ** (nn % 16))[:, None], jnp.bfloat16)
    ut = jnp.asarray(ww[:, None] <= ww[None, :], jnp.bfloat16)
    beta, a2, pw2 = _ball_query(pT, np16, seg, p2m, ut)
    gidx_flat = _sc_ballq_fix(pw2.reshape(B * M, _WP),
                              a2.reshape(B * M, _WP),
                              beta.reshape(B * M, NSAMPLE))
    gidx = gidx_flat.reshape(B, M, NSAMPLE)

    iflat_g = (gidx + offs[:, :, None]).reshape(-1)
    gx_flat = _sc_gather_rows(p16, iflat_g, 16, 2048)
    fj_flat = _sc_gather_rows(f1t, iflat_g, OUT_C, 1024)

    h1p, t1, u1 = _dp_mm_stats(gx_flat, np16_flat, W2_0, 8192)
    t2, u2 = _stats_only(h1p, t1, u1, r2(g2_0), r2(b2_0), W2_1, 8192)
    h3p, t3, u3 = _bn2_mm_stats(h1p, t1, u1, r2(g2_0), r2(b2_0), W2_1,
                                t2, u2, r2(g2_1), r2(b2_1), W2_2, 8192)
    pe_flat, fout_flat = _pe_max(h3p, t3, u3, r2(g2_2), r2(b2_2), fj_flat, 8192)

    pe_out = pe_flat.reshape(B, M, NSAMPLE, OUT_C).transpose(0, 3, 1, 2)
    fout = fout_flat.reshape(B, M, OUT_C).transpose(0, 2, 1)
    return new_p, fout, pe_out
